# baseline (speedup 1.0000x reference)
import functools

import jax
import jax.numpy as jnp
from jax import lax
from jax.experimental import pallas as pl
from jax.experimental.pallas import tpu as pltpu
from jax.experimental.pallas import tpu_sc as plsc

_IN = 512
_OUT = 16384
_K = 6
_BATCH = 1024
_NC = 2
_NS = 16
_NW = _NC * _NS
_BPW = _BATCH // _NW
_F = 1024
_NCHUNK = _OUT // _F
_G = _F // 16
_BU = 8


def _sc_body(x_hbm, c_hbm, o_hbm, xb, cvm, ovm):
    wid = lax.axis_index("s") * _NC + lax.axis_index("c")
    b0 = wid * _BPW
    pltpu.sync_copy(x_hbm.at[pl.ds(b0, _BPW), :], xb)

    def chunk_body(ci, carry):
        pltpu.sync_copy(c_hbm.at[:, pl.ds(ci * _F, _F)], cvm)

        def g_body(g, carry2):
            idx = [cvm[j, pl.ds(g * 16, 16)] for j in range(_K)]

            def b_body(bi, carry3):
                for db in range(_BU):
                    b = bi * _BU + db
                    bs = jnp.full((16,), b, jnp.int32)
                    acc = plsc.load_gather(xb, [bs, idx[0]])
                    for j in range(1, _K):
                        acc = acc + plsc.load_gather(xb, [bs, idx[j]])
                    ovm[b, pl.ds(g * 16, 16)] = acc
                return carry3

            lax.fori_loop(0, _BPW // _BU, b_body, 0)
            return carry2

        lax.fori_loop(0, _G, g_body, 0)
        pltpu.sync_copy(ovm, o_hbm.at[pl.ds(b0, _BPW), pl.ds(ci * _F, _F)])
        return carry

    lax.fori_loop(0, _NCHUNK, chunk_body, 0)


@jax.jit
def kernel(x, rows, cols, values):
    del rows, values
    c2 = cols.reshape(_OUT, _K).T

    mesh = plsc.VectorSubcoreMesh(core_axis_name="c", subcore_axis_name="s")
    f = pl.kernel(
        _sc_body,
        out_type=jax.ShapeDtypeStruct((_BATCH, _OUT), jnp.float32),
        mesh=mesh,
        compiler_params=pltpu.CompilerParams(
            use_tc_tiling_on_sc=False, needs_layout_passes=False
        ),
        scratch_types=[
            pltpu.VMEM((_BPW, _IN), jnp.float32),
            pltpu.VMEM((_K, _F), jnp.int32),
            pltpu.VMEM((_BPW, _F), jnp.float32),
        ],
    )
    return f(x, c2)


# speedup vs baseline: 1.5372x; 1.5372x over previous
import functools

import jax
import jax.numpy as jnp
from jax import lax
from jax.experimental import pallas as pl
from jax.experimental.pallas import tpu as pltpu
from jax.experimental.pallas import tpu_sc as plsc

_IN = 512
_OUT = 16384
_K = 6
_BATCH = 1024
_NC = 2
_NS = 16
_NW = _NC * _NS
_BPW = _BATCH // _NW
_F = 1024
_NCHUNK = _OUT // _F
_G = _F // 16
_BU = 8


def _sc_body(x_hbm, c_hbm, o_hbm, xb, cvm, ovm, csem, osem):
    wid = lax.axis_index("s") * _NC + lax.axis_index("c")
    b0 = wid * _BPW
    pltpu.sync_copy(x_hbm.at[pl.ds(b0 * _IN, _BPW * _IN)], xb)

    def c_copy(ci, buf):
        return pltpu.make_async_copy(
            c_hbm.at[:, pl.ds(ci * _F, _F)], cvm.at[buf], csem.at[buf]
        )

    def o_copy(ci, buf):
        return pltpu.make_async_copy(
            ovm.at[buf], o_hbm.at[pl.ds(b0, _BPW), pl.ds(ci * _F, _F)], osem.at[buf]
        )

    def fire_c(ci, buf):
        c_copy(ci, buf).start()

    def fire_o(ci, buf):
        o_copy(ci, buf).start()

    fire_c(0, 0)
    fire_c(1, 1)

    def chunk(ci, buf):
        c_copy(ci, buf).wait()

        @pl.when(ci >= 2)
        def _():
            o_copy(ci - 2, buf).wait()

        cb = cvm.at[buf]
        ob = ovm.at[buf]

        def g_body(g, carry):
            idx = [cb[j, pl.ds(g * 16, 16)] for j in range(_K)]

            @plsc.parallel_loop(0, _BPW, step=1, unroll=_BU)
            def b_body(b):
                bs = jnp.full((16,), b * _IN, jnp.int32)
                v = [plsc.load_gather(xb, [idx[j] + bs]) for j in range(_K)]
                ob[b, pl.ds(g * 16, 16)] = ((v[0] + v[1]) + (v[2] + v[3])) + (
                    v[4] + v[5]
                )

            return carry

        lax.fori_loop(0, _G, g_body, 0)
        fire_o(ci, buf)

        @pl.when(ci + 2 < _NCHUNK)
        def _():
            fire_c(ci + 2, buf)

    def super_chunk(sc, carry):
        chunk(2 * sc, 0)
        chunk(2 * sc + 1, 1)
        return carry

    lax.fori_loop(0, _NCHUNK // 2, super_chunk, 0)
    o_copy(_NCHUNK - 2, 0).wait()
    o_copy(_NCHUNK - 1, 1).wait()


@jax.jit
def kernel(x, rows, cols, values):
    del rows, values
    c2 = cols.reshape(_OUT, _K).T
    x1 = x.reshape(-1)

    mesh = plsc.VectorSubcoreMesh(core_axis_name="c", subcore_axis_name="s")
    f = pl.kernel(
        _sc_body,
        out_type=jax.ShapeDtypeStruct((_BATCH, _OUT), jnp.float32),
        mesh=mesh,
        compiler_params=pltpu.CompilerParams(
            use_tc_tiling_on_sc=False, needs_layout_passes=False
        ),
        scratch_types=[
            pltpu.VMEM((_BPW * _IN,), jnp.float32),
            pltpu.VMEM((2, _K, _F), jnp.int32),
            pltpu.VMEM((2, _BPW, _F), jnp.float32),
            pltpu.SemaphoreType.DMA((2,)),
            pltpu.SemaphoreType.DMA((2,)),
        ],
    )
    return f(x1, c2)


# speedup vs baseline: 1.7899x; 1.1644x over previous
import functools

import jax
import jax.numpy as jnp
from jax import lax
from jax.experimental import pallas as pl
from jax.experimental.pallas import tpu as pltpu
from jax.experimental.pallas import tpu_sc as plsc

_IN = 512
_OUT = 16384
_K = 6
_BATCH = 1024
_NC = 2
_NS = 16
_NW = _NC * _NS
_BPW = _BATCH // _NW
_F = 1024
_NCHUNK = _OUT // _F
_G = _F // 16
_BU = 8


def _sc_body(x_hbm, c_hbm, o_hbm, xb, cvm, ovm, csem, osem):
    wid = lax.axis_index("s") * _NC + lax.axis_index("c")
    b0 = wid * _BPW
    pltpu.sync_copy(x_hbm.at[pl.ds(b0 * _IN, _BPW * _IN)], xb)

    def c_copy(ci, buf):
        return pltpu.make_async_copy(
            c_hbm.at[:, pl.ds(ci * _F, _F)], cvm.at[buf], csem.at[buf]
        )

    def o_copy(ci, buf):
        return pltpu.make_async_copy(
            ovm.at[buf], o_hbm.at[pl.ds(b0, _BPW), pl.ds(ci * _F, _F)], osem.at[buf]
        )

    def fire_c(ci, buf):
        c_copy(ci, buf).start()

    def fire_o(ci, buf):
        o_copy(ci, buf).start()

    fire_c(0, 0)
    fire_c(1, 1)

    def chunk(ci, buf):
        c_copy(ci, buf).wait()

        @pl.when(ci >= 2)
        def _():
            o_copy(ci - 2, buf).wait()

        cb = cvm.at[buf]
        ob = ovm.at[buf]

        @plsc.parallel_loop(0, _G, step=1, unroll=2)
        def g_body(g):
            idx = [cb[j, pl.ds(g * 16, 16)] for j in range(_K)]
            for b in range(_BPW):
                bs = jnp.full((16,), b * _IN, jnp.int32)
                v = [plsc.load_gather(xb, [idx[j] + bs]) for j in range(_K)]
                ob[b, pl.ds(g * 16, 16)] = ((v[0] + v[1]) + (v[2] + v[3])) + (
                    v[4] + v[5]
                )
        fire_o(ci, buf)

        @pl.when(ci + 2 < _NCHUNK)
        def _():
            fire_c(ci + 2, buf)

    def super_chunk(sc, carry):
        chunk(2 * sc, 0)
        chunk(2 * sc + 1, 1)
        return carry

    lax.fori_loop(0, _NCHUNK // 2, super_chunk, 0)
    o_copy(_NCHUNK - 2, 0).wait()
    o_copy(_NCHUNK - 1, 1).wait()


@jax.jit
def kernel(x, rows, cols, values):
    del rows, values
    c2 = cols.reshape(_OUT, _K).T
    x1 = x.reshape(-1)

    mesh = plsc.VectorSubcoreMesh(core_axis_name="c", subcore_axis_name="s")
    f = pl.kernel(
        _sc_body,
        out_type=jax.ShapeDtypeStruct((_BATCH, _OUT), jnp.float32),
        mesh=mesh,
        compiler_params=pltpu.CompilerParams(
            use_tc_tiling_on_sc=False, needs_layout_passes=False
        ),
        scratch_types=[
            pltpu.VMEM((_BPW * _IN,), jnp.float32),
            pltpu.VMEM((2, _K, _F), jnp.int32),
            pltpu.VMEM((2, _BPW, _F), jnp.float32),
            pltpu.SemaphoreType.DMA((2,)),
            pltpu.SemaphoreType.DMA((2,)),
        ],
    )
    return f(x1, c2)


# speedup vs baseline: 2.0054x; 1.1204x over previous
import functools

import jax
import jax.numpy as jnp
from jax import lax
from jax.experimental import pallas as pl
from jax.experimental.pallas import tpu as pltpu
from jax.experimental.pallas import tpu_sc as plsc

_IN = 512
_OUT = 16384
_K = 6
_BATCH = 1024
_NC = 2
_NS = 16
_NW = _NC * _NS
_BPW = _BATCH // _NW
_F = 1024
_NCHUNK = _OUT // _F
_G = _F // 16
_BU = 8


def _sc_body(x_hbm, c_hbm, o_hbm, xb, cvm, ovm, csem, osem):
    wid = lax.axis_index("s") * _NC + lax.axis_index("c")
    b0 = wid * _BPW
    pltpu.sync_copy(x_hbm.at[pl.ds(b0, _BPW), :], xb)

    def c_copy(ci, buf):
        return pltpu.make_async_copy(
            c_hbm.at[:, pl.ds(ci * _F, _F)], cvm.at[buf], csem.at[buf]
        )

    def o_copy(ci, buf):
        return pltpu.make_async_copy(
            ovm.at[buf], o_hbm.at[pl.ds(b0, _BPW), pl.ds(ci * _F, _F)], osem.at[buf]
        )

    def fire_c(ci, buf):
        c_copy(ci, buf).start()

    def fire_o(ci, buf):
        o_copy(ci, buf).start()

    fire_c(0, 0)
    fire_c(1, 1)

    def chunk(ci, buf):
        c_copy(ci, buf).wait()

        @pl.when(ci >= 2)
        def _():
            o_copy(ci - 2, buf).wait()

        cb = cvm.at[buf]
        ob = ovm.at[buf]

        @plsc.parallel_loop(0, _G, step=1, unroll=2)
        def g_body(g):
            idx = [cb[j, pl.ds(g * 16, 16)] for j in range(_K)]
            for b in range(_BPW):
                row = xb.at[b]
                v = [plsc.load_gather(row, [idx[j]]) for j in range(_K)]
                ob[b, pl.ds(g * 16, 16)] = ((v[0] + v[1]) + (v[2] + v[3])) + (
                    v[4] + v[5]
                )
        fire_o(ci, buf)

        @pl.when(ci + 2 < _NCHUNK)
        def _():
            fire_c(ci + 2, buf)

    def super_chunk(sc, carry):
        chunk(2 * sc, 0)
        chunk(2 * sc + 1, 1)
        return carry

    lax.fori_loop(0, _NCHUNK // 2, super_chunk, 0)
    o_copy(_NCHUNK - 2, 0).wait()
    o_copy(_NCHUNK - 1, 1).wait()


@jax.jit
def kernel(x, rows, cols, values):
    del rows, values
    c2 = cols.reshape(_OUT, _K).T

    mesh = plsc.VectorSubcoreMesh(core_axis_name="c", subcore_axis_name="s")
    f = pl.kernel(
        _sc_body,
        out_type=jax.ShapeDtypeStruct((_BATCH, _OUT), jnp.float32),
        mesh=mesh,
        compiler_params=pltpu.CompilerParams(
            use_tc_tiling_on_sc=False, needs_layout_passes=False
        ),
        scratch_types=[
            pltpu.VMEM((_BPW, _IN), jnp.float32),
            pltpu.VMEM((2, _K, _F), jnp.int32),
            pltpu.VMEM((2, _BPW, _F), jnp.float32),
            pltpu.SemaphoreType.DMA((2,)),
            pltpu.SemaphoreType.DMA((2,)),
        ],
    )
    return f(x, c2)
